# Initial kernel scaffold; baseline (speedup 1.0000x reference)
import jax
import jax.numpy as jnp
from jax.experimental import pallas as pl


def kernel(x, params):
    raise NotImplementedError("write your pallas kernel here")


# speedup vs baseline: 18.8824x; 18.8824x over previous
import math

import jax
import jax.numpy as jnp
from jax.experimental import pallas as pl

_CFGS = [[1, 16, 1, 1], [6, 24, 2, 2], [6, 32, 3, 2], [6, 64, 4, 2],
         [6, 96, 3, 1], [6, 160, 3, 2], [6, 320, 1, 1]]
_NUM_CLASSES = 21
_TOPK = 200
_IOU_THR = 0.5


def _make_specs():
    specs = [('conv', 3, 32, 3, 2, 1, 1)]
    in_ch = 32
    for t, c, n, s in _CFGS:
        for i in range(n):
            stride = s if i == 0 else 1
            specs.append(('ir', in_ch, c, stride, t))
            in_ch = c
    specs.append(('conv', in_ch, 1280, 1, 1, 0, 1))
    return specs


def _conv2d(x, w, stride, padding, groups=1):
    return jax.lax.conv_general_dilated(
        x, w, (stride, stride), [(padding, padding), (padding, padding)],
        dimension_numbers=('NCHW', 'OIHW', 'NCHW'), feature_group_count=groups)


def _bn(x, g, b, eps=1e-5):
    mean = jnp.mean(x, axis=(0, 2, 3), keepdims=True)
    var = jnp.var(x, axis=(0, 2, 3), keepdims=True)
    xn = (x - mean) / jnp.sqrt(var + eps)
    return xn * g[None, :, None, None] + b[None, :, None, None]


def _relu6(x):
    return jnp.clip(x, 0.0, 6.0)


def _ssd_forward(x, params):
    specs = _make_specs()
    feats = []
    for li, spec in enumerate(specs):
        if spec[0] == 'conv':
            _, ci, co, ks, st, pd, gr = spec
            x = _relu6(_bn(_conv2d(x, params[f'l{li}_w'], st, pd, gr),
                           params[f'l{li}_g'], params[f'l{li}_b']))
        else:
            _, ci, co, st, t = spec
            hid = ci * t
            h = x
            if t != 1:
                h = _relu6(_bn(_conv2d(h, params[f'l{li}_pw1_w'], 1, 0),
                               params[f'l{li}_pw1_g'], params[f'l{li}_pw1_b']))
            h = _relu6(_bn(_conv2d(h, params[f'l{li}_dw_w'], st, 1, groups=hid),
                           params[f'l{li}_dw_g'], params[f'l{li}_dw_b']))
            h = _bn(_conv2d(h, params[f'l{li}_pw2_w'], 1, 0),
                    params[f'l{li}_pw2_g'], params[f'l{li}_pw2_b'])
            if st == 1 and ci == co:
                h = x + h
            x = h
        if li == 13 or li == 17:
            feats.append(x)
    loc_list, cls_list = [], []
    for i, f in enumerate(feats):
        lp = _conv2d(f, params[f'loc{i}_w'], 1, 1) + params[f'loc{i}_b2'][None, :, None, None]
        cp = _conv2d(f, params[f'cls{i}_w'], 1, 1) + params[f'cls{i}_b2'][None, :, None, None]
        loc_list.append(jnp.transpose(lp, (0, 2, 3, 1)).reshape(lp.shape[0], -1))
        cls_list.append(jnp.transpose(cp, (0, 2, 3, 1)).reshape(cp.shape[0], -1))
    B = x.shape[0]
    loc = jnp.concatenate(loc_list, axis=1).reshape(B, -1, 4)
    cls = jnp.concatenate(cls_list, axis=1).reshape(B, -1, _NUM_CLASSES)
    return loc, cls


def _nms_kernel(boxes_ref, cls_ref, boxes_out_ref, labels_out_ref,
                scores_out_ref, *, n_valid, topk_pad):
    B = boxes_ref.shape[1]
    Np = boxes_ref.shape[2]
    C = cls_ref.shape[0]
    neg_inf = jnp.float32(-jnp.inf)

    x1 = boxes_ref[0]
    y1 = boxes_ref[1]
    x2 = boxes_ref[2]
    y2 = boxes_ref[3]
    areas = (x2 - x1) * (y2 - y1)

    scores = cls_ref[0]
    labels = jnp.zeros((B, Np), jnp.int32)
    for k in range(1, C):
        cur = cls_ref[k]
        better = cur > scores
        scores = jnp.where(better, cur, scores)
        labels = jnp.where(better, k, labels)

    col = jax.lax.broadcasted_iota(jnp.int32, (B, Np), 1)
    colk = jax.lax.broadcasted_iota(jnp.int32, (B, topk_pad), 1)
    valid0 = (col < n_valid).astype(jnp.int32)

    def body(t, carry):
        valid32, ax1, ay1, ax2, ay2, alab, asc = carry
        valid = valid32 != 0
        masked = jnp.where(valid, scores, neg_inf)
        mval = jnp.max(masked, axis=1, keepdims=True)
        any_valid = jnp.any(valid, axis=1, keepdims=True)
        i = jnp.min(jnp.where(masked == mval, col, Np), axis=1, keepdims=True)
        onehot = col == i
        zero = jnp.float32(0.0)
        bx1 = jnp.sum(jnp.where(onehot, x1, zero), axis=1, keepdims=True)
        by1 = jnp.sum(jnp.where(onehot, y1, zero), axis=1, keepdims=True)
        bx2 = jnp.sum(jnp.where(onehot, x2, zero), axis=1, keepdims=True)
        by2 = jnp.sum(jnp.where(onehot, y2, zero), axis=1, keepdims=True)
        bar = jnp.sum(jnp.where(onehot, areas, zero), axis=1, keepdims=True)
        bsc = jnp.sum(jnp.where(onehot, scores, zero), axis=1, keepdims=True)
        blab = jnp.sum(jnp.where(onehot, labels, 0), axis=1, keepdims=True)

        xx1 = jnp.maximum(bx1, x1)
        yy1 = jnp.maximum(by1, y1)
        xx2 = jnp.minimum(bx2, x2)
        yy2 = jnp.minimum(by2, y2)
        w = jnp.maximum(xx2 - xx1, zero)
        h = jnp.maximum(yy2 - yy1, zero)
        inter = w * h
        union = bar + areas - inter
        iou = inter / union
        valid32 = (valid & (iou <= _IOU_THR) & (col != i)).astype(jnp.int32)

        m = jnp.where(any_valid, jnp.float32(1.0), zero)
        mi = jnp.where(any_valid, 1, 0)
        sel = colk == t
        ax1 = jnp.where(sel, bx1 * m, ax1)
        ay1 = jnp.where(sel, by1 * m, ay1)
        ax2 = jnp.where(sel, bx2 * m, ax2)
        ay2 = jnp.where(sel, by2 * m, ay2)
        alab = jnp.where(sel, blab * mi, alab)
        asc = jnp.where(sel, bsc * m, asc)
        return valid32, ax1, ay1, ax2, ay2, alab, asc

    zf = jnp.zeros((B, topk_pad), jnp.float32)
    zi = jnp.zeros((B, topk_pad), jnp.int32)
    carry = (valid0, zf, zf, zf, zf, zi, zf)
    _, ax1, ay1, ax2, ay2, alab, asc = jax.lax.fori_loop(0, _TOPK, body, carry)

    boxes_out_ref[0] = ax1[:, :_TOPK]
    boxes_out_ref[1] = ay1[:, :_TOPK]
    boxes_out_ref[2] = ax2[:, :_TOPK]
    boxes_out_ref[3] = ay2[:, :_TOPK]
    labels_out_ref[...] = alab[:, :_TOPK]
    scores_out_ref[...] = asc[:, :_TOPK]


def kernel(x, params):
    loc, cls = _ssd_forward(x, params)
    B, N, _ = loc.shape
    Np = ((N + 127) // 128) * 128
    topk_pad = ((_TOPK + 127) // 128) * 128

    boxes_t = jnp.transpose(loc, (2, 0, 1))
    boxes_t = jnp.pad(boxes_t, ((0, 0), (0, 0), (0, Np - N)))
    cls_t = jnp.transpose(cls, (2, 0, 1))
    cls_t = jnp.pad(cls_t, ((0, 0), (0, 0), (0, Np - N)),
                    constant_values=-jnp.inf)

    import functools
    kern = functools.partial(_nms_kernel, n_valid=N, topk_pad=topk_pad)
    boxes_o, labels_o, scores_o = pl.pallas_call(
        kern,
        out_shape=(
            jax.ShapeDtypeStruct((4, B, _TOPK), jnp.float32),
            jax.ShapeDtypeStruct((B, _TOPK), jnp.int32),
            jax.ShapeDtypeStruct((B, _TOPK), jnp.float32),
        ),
    )(boxes_t, cls_t)

    boxes_out = jnp.transpose(boxes_o, (1, 2, 0))
    return boxes_out, labels_o, scores_o
